# Initial kernel scaffold; baseline (speedup 1.0000x reference)
import jax
import jax.numpy as jnp
from jax.experimental import pallas as pl


def kernel(x, adj_indices, adj_values, W, b):
    raise NotImplementedError("write your pallas kernel here")


# speedup vs baseline: 2.7062x; 2.7062x over previous
import functools

import jax
import jax.numpy as jnp
from jax import lax
from jax.experimental import pallas as pl
from jax.experimental.pallas import tpu as pltpu
from jax.experimental.pallas import tpu_sc as plsc

_NC = 2
_NS = 16
_NW = _NC * _NS
_L = 16

_CHUNK = 80


def _matmul(x, W):
    n, d_in = x.shape
    d_out = W.shape[1]
    blk = 1000

    def body(x_ref, w_ref, o_ref):
        o_ref[...] = jnp.dot(x_ref[...], w_ref[...],
                             preferred_element_type=jnp.float32)

    return pl.pallas_call(
        body,
        grid=(n // blk,),
        in_specs=[
            pl.BlockSpec((blk, d_in), lambda i: (i, 0)),
            pl.BlockSpec((d_in, d_out), lambda i: (0, 0)),
        ],
        out_specs=pl.BlockSpec((blk, d_out), lambda i: (i, 0)),
        out_shape=jax.ShapeDtypeStruct((n, d_out), jnp.float32),
    )(x, W)


def _combine(partials, b):
    _, n, d = partials.shape
    blk = 1000

    def body(p_ref, b_ref, o_ref):
        o_ref[...] = p_ref[0] + p_ref[1] + b_ref[...]

    return pl.pallas_call(
        body,
        grid=(n // blk,),
        in_specs=[
            pl.BlockSpec((2, blk, d), lambda i: (0, i, 0)),
            pl.BlockSpec((1, d), lambda i: (0, 0)),
        ],
        out_specs=pl.BlockSpec((blk, d), lambda i: (i, 0)),
        out_shape=jax.ShapeDtypeStruct((n, d), jnp.float32),
    )(partials, b)


def _make_sc_scatter(n_nodes, n_edges, d):
    e_w = n_edges // _NW
    nchunk = e_w // _CHUNK
    rows_t = 624
    extra0 = _NS * rows_t
    extra = n_nodes - extra0
    zr = 16
    wb = 208
    groups = d // _L

    mesh = plsc.VectorSubcoreMesh(core_axis_name="c", subcore_axis_name="s")

    @functools.partial(
        pl.kernel,
        out_type=jax.ShapeDtypeStruct((_NC, n_nodes, d), jnp.float32),
        mesh=mesh,
        scratch_types=[
            pltpu.VMEM((_CHUNK,), jnp.int32),
            pltpu.VMEM((_CHUNK,), jnp.int32),
            pltpu.VMEM((_CHUNK * _L,), jnp.float32),
            pltpu.VMEM((_CHUNK, d), jnp.float32),
            pltpu.VMEM((zr, d), jnp.float32),
            pltpu.VMEM_SHARED((n_nodes, d), jnp.float32),
            pltpu.SemaphoreType.DMA,
        ],
    )
    def sc_scatter(rows_hbm, cols_hbm, vals_hbm, sup_hbm, out_hbm,
                   colb, rowb, valb, gbuf, zbuf, acc, sem):
        c = lax.axis_index("c")
        s = lax.axis_index("s")
        wid = c * _NS + s

        def zrow(i, carry):
            for j in range(groups):
                zbuf[i, pl.ds(j * _L, _L)] = jnp.zeros((_L,), jnp.float32)
            return carry
        lax.fori_loop(0, zr, zrow, 0)
        r0 = s * rows_t

        def zcopy(k, carry):
            pltpu.sync_copy(zbuf, acc.at[pl.ds(r0 + k * zr, zr), :])
            return carry
        lax.fori_loop(0, rows_t // zr, zcopy, 0)

        @pl.when(s == _NS - 1)
        def _():
            pltpu.sync_copy(zbuf, acc.at[pl.ds(extra0, extra), :])

        plsc.subcore_barrier()

        base = wid * e_w

        def chunk_body(ci, carry):
            off = base + ci * _CHUNK
            pltpu.sync_copy(rows_hbm.at[pl.ds(off, _CHUNK)], rowb)
            pltpu.sync_copy(cols_hbm.at[pl.ds(off, _CHUNK)], colb)
            pltpu.sync_copy(vals_hbm.at[pl.ds(off * _L, _CHUNK * _L)], valb)
            pltpu.async_copy(sup_hbm.at[colb], gbuf, sem).wait()

            def erow(i, icarry):
                vb = valb[pl.ds(i * _L, _L)]
                for j in range(groups):
                    gbuf[i, pl.ds(j * _L, _L)] = gbuf[i, pl.ds(j * _L, _L)] * vb
                return icarry
            lax.fori_loop(0, _CHUNK, erow, 0)

            pltpu.sync_copy(gbuf, acc.at[rowb], add=True)
            return carry
        lax.fori_loop(0, nchunk, chunk_body, 0)

        plsc.subcore_barrier()

        for k in range(rows_t // wb):
            pltpu.sync_copy(acc.at[pl.ds(r0 + k * wb, wb), :],
                            out_hbm.at[c, pl.ds(r0 + k * wb, wb), :])

        @pl.when(s == _NS - 1)
        def _():
            pltpu.sync_copy(acc.at[pl.ds(extra0, extra), :],
                            out_hbm.at[c, pl.ds(extra0, extra), :])

    return sc_scatter


def kernel(x, adj_indices, adj_values, W, b):
    n, _ = x.shape
    d = W.shape[1]
    e = adj_values.shape[0]

    support = _matmul(x, W)
    rows = adj_indices[0]
    cols = adj_indices[1]
    vals_rep = jnp.repeat(adj_values, _L)
    partials = _make_sc_scatter(n, e, d)(rows, cols, vals_rep, support)
    return _combine(partials, b)


# speedup vs baseline: 4.8760x; 1.8018x over previous
import functools

import jax
import jax.numpy as jnp
from jax import lax
from jax.experimental import pallas as pl
from jax.experimental.pallas import tpu as pltpu
from jax.experimental.pallas import tpu_sc as plsc

_NC = 2
_NS = 16
_NW = _NC * _NS
_L = 16

_CHUNK = 80


def _matmul(x, W):
    n, d_in = x.shape
    d_out = W.shape[1]
    blk = 1000

    def body(x_ref, w_ref, o_ref):
        o_ref[...] = jnp.dot(x_ref[...], w_ref[...],
                             preferred_element_type=jnp.float32)

    return pl.pallas_call(
        body,
        grid=(n // blk,),
        in_specs=[
            pl.BlockSpec((blk, d_in), lambda i: (i, 0)),
            pl.BlockSpec((d_in, d_out), lambda i: (0, 0)),
        ],
        out_specs=pl.BlockSpec((blk, d_out), lambda i: (i, 0)),
        out_shape=jax.ShapeDtypeStruct((n, d_out), jnp.float32),
    )(x, W)


def _combine(partials, b):
    _, n, d = partials.shape
    blk = 2000

    def body(p_ref, b_ref, o_ref):
        o_ref[...] = p_ref[0] + p_ref[1] + b_ref[...]

    return pl.pallas_call(
        body,
        grid=(n // blk,),
        in_specs=[
            pl.BlockSpec((2, blk, d), lambda i: (0, i, 0)),
            pl.BlockSpec((1, d), lambda i: (0, 0)),
        ],
        out_specs=pl.BlockSpec((blk, d), lambda i: (i, 0)),
        out_shape=jax.ShapeDtypeStruct((n, d), jnp.float32),
    )(partials, b)


def _make_sc_scatter(n_nodes, n_edges, d):
    e_w = n_edges // _NW
    nchunk = e_w // _CHUNK
    rows_t = 624
    extra0 = _NS * rows_t
    extra = n_nodes - extra0
    zr = 16
    nz = rows_t // zr
    wb = 208
    groups = d // _L

    mesh = plsc.VectorSubcoreMesh(core_axis_name="c", subcore_axis_name="s")

    @functools.partial(
        pl.kernel,
        out_type=jax.ShapeDtypeStruct((_NC, n_nodes, d), jnp.float32),
        mesh=mesh,
        scratch_types=[
            [pltpu.VMEM((_CHUNK,), jnp.int32) for _ in range(3)],
            [pltpu.VMEM((_CHUNK,), jnp.int32) for _ in range(3)],
            [pltpu.VMEM((_CHUNK * _L,), jnp.float32) for _ in range(3)],
            [pltpu.VMEM((_CHUNK, d), jnp.float32) for _ in range(2)],
            pltpu.VMEM((zr, d), jnp.float32),
            pltpu.VMEM_SHARED((n_nodes, d), jnp.float32),
            [pltpu.SemaphoreType.DMA for _ in range(3)],
            [pltpu.SemaphoreType.DMA for _ in range(2)],
            [pltpu.SemaphoreType.DMA for _ in range(2)],
            pltpu.SemaphoreType.DMA,
        ],
    )
    def sc_scatter(rows_hbm, cols_hbm, vals_hbm, sup_hbm, out_hbm,
                   colb, rowb, valb, gbuf, zbuf, acc,
                   isem, gsem, ssem, wsem):
        c = lax.axis_index("c")
        s = lax.axis_index("s")
        wid = c * _NS + s
        base = wid * e_w

        def zrow(i, carry):
            for j in range(groups):
                zbuf[i, pl.ds(j * _L, _L)] = jnp.zeros((_L,), jnp.float32)
            return carry
        lax.fori_loop(0, zr, zrow, 0)
        r0 = s * rows_t

        def zfire(k, carry):
            pltpu.async_copy(zbuf, acc.at[pl.ds(r0 + k * zr, zr), :], wsem)
            return carry
        lax.fori_loop(0, nz, zfire, 0)

        @pl.when(s == _NS - 1)
        def _():
            pltpu.async_copy(zbuf, acc.at[pl.ds(extra0, extra), :], wsem)

        def zdrain(k, carry):
            pltpu.make_async_copy(zbuf, acc.at[pl.ds(r0 + k * zr, zr), :],
                                  wsem).wait()
            return carry
        lax.fori_loop(0, nz, zdrain, 0)

        @pl.when(s == _NS - 1)
        def _():
            pltpu.make_async_copy(zbuf, acc.at[pl.ds(extra0, extra), :],
                                  wsem).wait()

        plsc.subcore_barrier()

        def start_idx(ci, r):
            off = base + ci * _CHUNK
            pltpu.async_copy(rows_hbm.at[pl.ds(off, _CHUNK)], rowb[r], isem[r])
            pltpu.async_copy(cols_hbm.at[pl.ds(off, _CHUNK)], colb[r], isem[r])
            pltpu.async_copy(vals_hbm.at[pl.ds(off * _L, _CHUNK * _L)],
                             valb[r], isem[r])

        def wait_idx(r):
            pltpu.make_async_copy(rows_hbm.at[pl.ds(0, _CHUNK)], rowb[r],
                                  isem[r]).wait()
            pltpu.make_async_copy(cols_hbm.at[pl.ds(0, _CHUNK)], colb[r],
                                  isem[r]).wait()
            pltpu.make_async_copy(vals_hbm.at[pl.ds(0, _CHUNK * _L)], valb[r],
                                  isem[r]).wait()

        def start_gather(r, p):
            pltpu.async_copy(sup_hbm.at[colb[r]], gbuf[p], gsem[p])

        def wait_gather(r, p):
            pltpu.make_async_copy(sup_hbm.at[colb[r]], gbuf[p],
                                  gsem[p]).wait()

        def start_scatter(r, p):
            pltpu.async_copy(gbuf[p], acc.at[rowb[r]], ssem[p], add=True)

        def wait_scatter(r, p):
            pltpu.make_async_copy(gbuf[p], acc.at[rowb[r]], ssem[p]).wait()

        def multiply(r, p):
            gb = gbuf[p]
            vb_ref = valb[r]

            @plsc.parallel_loop(0, _CHUNK, step=1, unroll=4)
            def _(i):
                vb = vb_ref[pl.ds(i * _L, _L)]
                for j in range(groups):
                    gb[i, pl.ds(j * _L, _L)] = gb[i, pl.ds(j * _L, _L)] * vb

        def emit_chunk(ci, j, first=False, last1=False, last2=False):
            p = j % 2
            r = j % 3
            if not last1:
                wait_idx((r + 1) % 3)
                if not first:
                    wait_scatter((r + 2) % 3, 1 - p)
                start_gather((r + 1) % 3, 1 - p)
            wait_gather(r, p)
            if not last2:
                start_idx(ci + 2, (r + 2) % 3)
            multiply(r, p)
            start_scatter(r, p)

        start_idx(0, 0)
        start_idx(1, 1)
        wait_idx(0)
        start_gather(0, 0)
        emit_chunk(0, 0, first=True)

        def six(k, carry):
            cb = 1 + k * 6
            for j in range(6):
                emit_chunk(cb + j, 1 + j)
            return carry
        lax.fori_loop(0, (nchunk - 5) // 6, six, 0)

        for ci in range(nchunk - 4, nchunk):
            emit_chunk(ci, ci, last1=(ci == nchunk - 1),
                       last2=(ci >= nchunk - 2))

        wait_scatter((nchunk - 2) % 3, (nchunk - 2) % 2)
        wait_scatter((nchunk - 1) % 3, (nchunk - 1) % 2)

        plsc.subcore_barrier()

        for k in range(rows_t // wb):
            pltpu.async_copy(acc.at[pl.ds(r0 + k * wb, wb), :],
                             out_hbm.at[c, pl.ds(r0 + k * wb, wb), :], wsem)

        @pl.when(s == _NS - 1)
        def _():
            pltpu.async_copy(acc.at[pl.ds(extra0, extra), :],
                             out_hbm.at[c, pl.ds(extra0, extra), :], wsem)

        for k in range(rows_t // wb):
            pltpu.make_async_copy(acc.at[pl.ds(r0 + k * wb, wb), :],
                                  out_hbm.at[c, pl.ds(r0 + k * wb, wb), :],
                                  wsem).wait()

        @pl.when(s == _NS - 1)
        def _():
            pltpu.make_async_copy(acc.at[pl.ds(extra0, extra), :],
                                  out_hbm.at[c, pl.ds(extra0, extra), :],
                                  wsem).wait()

    return sc_scatter


def kernel(x, adj_indices, adj_values, W, b):
    n, _ = x.shape
    d = W.shape[1]
    e = adj_values.shape[0]

    support = _matmul(x, W)
    rows = adj_indices[0]
    cols = adj_indices[1]
    vals_rep = jnp.repeat(adj_values, _L)
    partials = _make_sc_scatter(n, e, d)(rows, cols, vals_rep, support)
    return _combine(partials, b)


# speedup vs baseline: 11.4845x; 2.3553x over previous
import functools

import jax
import jax.numpy as jnp
from jax import lax
from jax.experimental import pallas as pl
from jax.experimental.pallas import tpu as pltpu
from jax.experimental.pallas import tpu_sc as plsc

_NC = 2
_NS = 16
_NW = _NC * _NS
_L = 16

_CHUNK = 80


def _matmul(x, W):
    n, d_in = x.shape
    d_out = W.shape[1]
    blk = 1000

    def body(x_ref, w_ref, o_ref):
        o_ref[...] = jnp.dot(x_ref[...], w_ref[...],
                             preferred_element_type=jnp.float32)

    return pl.pallas_call(
        body,
        grid=(n // blk,),
        in_specs=[
            pl.BlockSpec((blk, d_in), lambda i: (i, 0)),
            pl.BlockSpec((d_in, d_out), lambda i: (0, 0)),
        ],
        out_specs=pl.BlockSpec((blk, d_out), lambda i: (i, 0)),
        out_shape=jax.ShapeDtypeStruct((n, d_out), jnp.float32),
    )(x, W)


def _combine(partials, b):
    _, n, d = partials.shape
    blk = 2000

    def body(p_ref, b_ref, o_ref):
        o_ref[...] = p_ref[0] + p_ref[1] + b_ref[...]

    return pl.pallas_call(
        body,
        grid=(n // blk,),
        in_specs=[
            pl.BlockSpec((2, blk, d), lambda i: (0, i, 0)),
            pl.BlockSpec((1, d), lambda i: (0, 0)),
        ],
        out_specs=pl.BlockSpec((blk, d), lambda i: (i, 0)),
        out_shape=jax.ShapeDtypeStruct((n, d), jnp.float32),
    )(partials, b)


def _make_sc_scatter(n_nodes, n_edges, d):
    e_w = n_edges // _NW
    nchunk = e_w // _CHUNK
    rows_t = 624
    extra0 = _NS * rows_t
    extra = n_nodes - extra0
    zr = 16
    nz = rows_t // zr
    wb = 208
    groups = d // _L

    mesh = plsc.VectorSubcoreMesh(core_axis_name="c", subcore_axis_name="s")

    @functools.partial(
        pl.kernel,
        out_type=jax.ShapeDtypeStruct((_NC, n_nodes, d), jnp.float32),
        mesh=mesh,
        scratch_types=[
            [pltpu.VMEM((_CHUNK,), jnp.int32) for _ in range(3)],
            [pltpu.VMEM((_CHUNK,), jnp.int32) for _ in range(3)],
            [pltpu.VMEM((_CHUNK,), jnp.int32) for _ in range(3)],
            [pltpu.VMEM((_CHUNK + _L,), jnp.float32) for _ in range(3)],
            [pltpu.VMEM((_CHUNK, d), jnp.float32) for _ in range(3)],
            pltpu.VMEM((zr, d), jnp.float32),
            pltpu.VMEM_SHARED((n_nodes, d), jnp.float32),
            [pltpu.SemaphoreType.DMA for _ in range(3)],
            [pltpu.SemaphoreType.DMA for _ in range(3)],
            [pltpu.SemaphoreType.DMA for _ in range(3)],
            pltpu.SemaphoreType.DMA,
        ],
    )
    def sc_scatter(rows_hbm, cols_hbm, vals_hbm, sup_hbm, out_hbm,
                   colb, rowb, rowsc, valb, gbuf, zbuf, acc,
                   isem, gsem, ssem, wsem):
        c = lax.axis_index("c")
        s = lax.axis_index("s")
        wid = c * _NS + s
        base = wid * e_w

        def zrow(i, carry):
            for j in range(groups):
                zbuf[i, pl.ds(j * _L, _L)] = jnp.zeros((_L,), jnp.float32)
            return carry
        lax.fori_loop(0, zr, zrow, 0)
        r0 = s * rows_t

        def zfire(k, carry):
            pltpu.async_copy(zbuf, acc.at[pl.ds(r0 + k * zr, zr), :], wsem)
            return carry
        lax.fori_loop(0, nz, zfire, 0)

        @pl.when(s == _NS - 1)
        def _():
            pltpu.async_copy(zbuf, acc.at[pl.ds(extra0, extra), :], wsem)

        def zdrain(k, carry):
            pltpu.make_async_copy(zbuf, acc.at[pl.ds(r0 + k * zr, zr), :],
                                  wsem).wait()
            return carry
        lax.fori_loop(0, nz, zdrain, 0)

        @pl.when(s == _NS - 1)
        def _():
            pltpu.make_async_copy(zbuf, acc.at[pl.ds(extra0, extra), :],
                                  wsem).wait()

        plsc.subcore_barrier()

        def start_idx(ci, r):
            off = base + ci * _CHUNK
            pltpu.async_copy(rows_hbm.at[pl.ds(off, _CHUNK)], rowb[r], isem[r])
            pltpu.async_copy(cols_hbm.at[pl.ds(off, _CHUNK)], colb[r], isem[r])
            pltpu.async_copy(vals_hbm.at[pl.ds(off, _CHUNK)],
                             valb[r].at[pl.ds(0, _CHUNK)], isem[r])

        def wait_idx(r):
            pltpu.make_async_copy(rows_hbm.at[pl.ds(0, _CHUNK)], rowb[r],
                                  isem[r]).wait()
            pltpu.make_async_copy(cols_hbm.at[pl.ds(0, _CHUNK)], colb[r],
                                  isem[r]).wait()
            pltpu.make_async_copy(vals_hbm.at[pl.ds(0, _CHUNK)],
                                  valb[r].at[pl.ds(0, _CHUNK)], isem[r]).wait()

        def start_gather(r, p):
            pltpu.async_copy(sup_hbm.at[colb[r]], gbuf[p], gsem[p])

        def wait_gather(r, p):
            pltpu.make_async_copy(sup_hbm.at[colb[r]], gbuf[p],
                                  gsem[p]).wait()

        def start_scatter(p):
            pltpu.async_copy(gbuf[p], acc.at[rowsc[p]], ssem[p], add=True)

        def wait_scatter(p):
            pltpu.make_async_copy(gbuf[p], acc.at[rowsc[p]], ssem[p]).wait()

        def copy_rows(p):
            for t in range(_CHUNK // _L):
                rowsc[p][pl.ds(t * _L, _L)] = rowb[p][pl.ds(t * _L, _L)]

        def multiply(r, p):
            gb = gbuf[p]
            vb_ref = valb[r]

            @plsc.parallel_loop(0, _CHUNK, step=1, unroll=8)
            def _(i):
                vseg = vb_ref[pl.ds(i, _L)]
                vb = jnp.full((_L,), vseg[0], jnp.float32)
                for j in range(groups):
                    gb[i, pl.ds(j * _L, _L)] = gb[i, pl.ds(j * _L, _L)] * vb

        def emit_chunk(ci, j, first2=False, last1=False, last2=False):
            p = j % 3
            if not last1:
                wait_idx((p + 1) % 3)
            if not first2:
                wait_scatter((p + 1) % 3)
            if not last1:
                start_gather((p + 1) % 3, (p + 1) % 3)
            wait_gather(p, p)
            if not last2:
                start_idx(ci + 2, (p + 2) % 3)
            copy_rows(p)
            multiply(p, p)
            start_scatter(p)

        start_idx(0, 0)
        start_idx(1, 1)
        wait_idx(0)
        start_gather(0, 0)
        emit_chunk(0, 0, first2=True)
        emit_chunk(1, 1, first2=True)

        def three(k, carry):
            cb = 2 + k * 3
            for j in range(3):
                emit_chunk(cb + j, 2 + j)
            return carry
        lax.fori_loop(0, (nchunk - 8) // 3, three, 0)

        for ci in range(nchunk - 6, nchunk):
            emit_chunk(ci, ci, last1=(ci == nchunk - 1),
                       last2=(ci >= nchunk - 2))

        wait_scatter((nchunk - 2) % 3)
        wait_scatter((nchunk - 1) % 3)

        plsc.subcore_barrier()

        for k in range(rows_t // wb):
            pltpu.async_copy(acc.at[pl.ds(r0 + k * wb, wb), :],
                             out_hbm.at[c, pl.ds(r0 + k * wb, wb), :], wsem)

        @pl.when(s == _NS - 1)
        def _():
            pltpu.async_copy(acc.at[pl.ds(extra0, extra), :],
                             out_hbm.at[c, pl.ds(extra0, extra), :], wsem)

        for k in range(rows_t // wb):
            pltpu.make_async_copy(acc.at[pl.ds(r0 + k * wb, wb), :],
                                  out_hbm.at[c, pl.ds(r0 + k * wb, wb), :],
                                  wsem).wait()

        @pl.when(s == _NS - 1)
        def _():
            pltpu.make_async_copy(acc.at[pl.ds(extra0, extra), :],
                                  out_hbm.at[c, pl.ds(extra0, extra), :],
                                  wsem).wait()

    return sc_scatter


def kernel(x, adj_indices, adj_values, W, b):
    n, _ = x.shape
    d = W.shape[1]
    e = adj_values.shape[0]

    support = _matmul(x, W)
    rows = adj_indices[0]
    cols = adj_indices[1]
    partials = _make_sc_scatter(n, e, d)(rows, cols, adj_values, support)
    return _combine(partials, b)
